# baseline (device time: 266041 ns/iter reference)
import jax
import jax.numpy as jnp
from jax import lax
from jax.experimental import pallas as pl
from jax.experimental.pallas import tpu as pltpu

N_DEV = 4
SQ = 2048
SKV = 2048
HQ = 8
DH = 128
DM = 1024
DQ = HQ * DH
SCALE = 0.08838834764831843
BLK = 64
QB = 256
NQB = SQ // QB
CH = SQ // N_DEV


def _attn_body(x_ref, wq_ref, k_ref, v_ref, wo_ref, out_ref, ctx_ref):
    for qb in range(NQB):
        rows = pl.ds(qb * QB, QB)
        q_all = jnp.dot(x_ref[rows, :], wq_ref[:, :],
                        preferred_element_type=jnp.float32)
        for h in range(HQ):
            qh = q_all[:, h * DH:(h + 1) * DH]
            kh = k_ref[h]
            vh = v_ref[h]
            s = lax.dot_general(qh, kh, (((1,), (1,)), ((), ())),
                                preferred_element_type=jnp.float32) * SCALE
            r = (lax.broadcasted_iota(jnp.int32, (QB, SKV), 0) // BLK
                 + qb * QB // BLK)
            c = lax.broadcasted_iota(jnp.int32, (QB, SKV), 1) // BLK
            mask = (r == c) | (c == 0) | (lax.rem(r + c, 3) == 0)
            s = jnp.where(mask, s, -1e9)
            m = jnp.max(s, axis=1, keepdims=True)
            w = jnp.exp(s - m)
            w = w / jnp.sum(w, axis=1, keepdims=True)
            ctx_ref[:, h * DH:(h + 1) * DH] = jnp.dot(
                w, vh, preferred_element_type=jnp.float32)
        out_ref[rows, :] = jnp.dot(ctx_ref[:, :], wo_ref[:, :],
                                   preferred_element_type=jnp.float32)


def _ar_body(p_ref, out_ref, rbuf, rs_send, rs_recv, ag_send, ag_recv):
    my = lax.axis_index("i")
    left = lax.rem(my + N_DEV - 1, N_DEV)
    right = lax.rem(my + 1, N_DEV)

    barrier = pltpu.get_barrier_semaphore()
    for nbr in (left, right):
        pl.semaphore_signal(barrier, inc=1, device_id=(nbr,),
                            device_id_type=pl.DeviceIdType.MESH)
    pl.semaphore_wait(barrier, 2)

    out_ref[:, :] = p_ref[:, :]

    for s in range(N_DEV - 1):
        cs = lax.rem(my - s + N_DEV, N_DEV)
        cr = lax.rem(my - s - 1 + N_DEV, N_DEV)
        rdma = pltpu.make_async_remote_copy(
            src_ref=out_ref.at[pl.ds(cs * CH, CH), :],
            dst_ref=rbuf.at[s],
            send_sem=rs_send.at[s],
            recv_sem=rs_recv.at[s],
            device_id=(right,),
            device_id_type=pl.DeviceIdType.MESH,
        )
        rdma.start()
        rdma.wait()
        out_ref[pl.ds(cr * CH, CH), :] = (
            out_ref[pl.ds(cr * CH, CH), :] + rbuf[s])

    for t in range(N_DEV - 1):
        cs = lax.rem(my + 1 - t + N_DEV, N_DEV)
        rdma = pltpu.make_async_remote_copy(
            src_ref=out_ref.at[pl.ds(cs * CH, CH), :],
            dst_ref=out_ref.at[pl.ds(cs * CH, CH), :],
            send_sem=ag_send.at[t],
            recv_sem=ag_recv.at[t],
            device_id=(right,),
            device_id_type=pl.DeviceIdType.MESH,
        )
        rdma.start()
        rdma.wait()


def kernel(x, Wq, K_ext, V_ext, Wo):
    i = lax.axis_index("i")
    x2 = x[0]
    wq = lax.dynamic_slice(Wq, (0, i * DQ), (DM, DQ))
    wo = lax.dynamic_slice(Wo, (i * DQ, 0), (DQ, DM))
    k = jnp.transpose(K_ext[0], (1, 0, 2))
    v = jnp.transpose(V_ext[0], (1, 0, 2))

    partial = pl.pallas_call(
        _attn_body,
        out_shape=jax.ShapeDtypeStruct((SQ, DM), jnp.float32),
        in_specs=[pl.BlockSpec(memory_space=pltpu.VMEM)] * 5,
        out_specs=pl.BlockSpec(memory_space=pltpu.VMEM),
        scratch_shapes=[pltpu.VMEM((QB, DQ), jnp.float32)],
    )(x2, wq, k, v, wo)

    out = pl.pallas_call(
        _ar_body,
        out_shape=jax.ShapeDtypeStruct((SQ, DM), jnp.float32),
        in_specs=[pl.BlockSpec(memory_space=pltpu.VMEM)],
        out_specs=pl.BlockSpec(memory_space=pltpu.VMEM),
        scratch_shapes=[
            pltpu.VMEM((N_DEV - 1, CH, DM), jnp.float32),
            pltpu.SemaphoreType.DMA((N_DEV - 1,)),
            pltpu.SemaphoreType.DMA((N_DEV - 1,)),
            pltpu.SemaphoreType.DMA((N_DEV - 1,)),
            pltpu.SemaphoreType.DMA((N_DEV - 1,)),
        ],
        compiler_params=pltpu.CompilerParams(collective_id=0),
    )(partial)

    return out[None]
